# baseline (speedup 1.0000x reference)
import functools

import jax
import jax.numpy as jnp
from jax import lax
from jax.experimental import pallas as pl
from jax.experimental.pallas import tpu as pltpu
from jax.experimental.pallas import tpu_sc as plsc

_CHUNK = 128
_PAIR = 2
_NC = 2
_NS = 16
_L = 16


def _sc_segment_sum(x, src2d, dst2d, n_pad):
    d = x.shape[1]
    nchunks = src2d.shape[0]
    rpt = n_pad // _NS
    nworkers = _NC * _NS
    cpt = nchunks // nworkers
    assert cpt * nworkers == nchunks

    mesh = plsc.VectorSubcoreMesh(core_axis_name="c", subcore_axis_name="s")

    @functools.partial(
        pl.kernel,
        out_type=(
            jax.ShapeDtypeStruct((_NC, n_pad, d), jnp.float32),
            jax.ShapeDtypeStruct((_NC * n_pad,), jnp.float32),
        ),
        mesh=mesh,
        scratch_types=[
            pltpu.VMEM((cpt, _CHUNK), jnp.int32),
            pltpu.VMEM((cpt, _CHUNK), jnp.int32),
            pltpu.VMEM((_CHUNK,), jnp.int32),
            pltpu.VMEM((_CHUNK,), jnp.int32),
            pltpu.VMEM((_CHUNK, d), jnp.float32),
            pltpu.VMEM((_CHUNK,), jnp.float32),
            pltpu.VMEM((-(-rpt // _L) * _L,), jnp.float32),
            pltpu.VMEM_SHARED((n_pad, d), jnp.float32),
            pltpu.VMEM_SHARED((n_pad,), jnp.float32),
            pltpu.SemaphoreType.DMA,
        ],
    )
    def sc_kernel(x_hbm, src_hbm, dst_hbm, s_out, c_out,
                  sall, dall, sidx, didx, rows, ones1, cbuf, s_sh, c_sh,
                  sem):
        c = lax.axis_index("c")
        s = lax.axis_index("s")
        wid = s * _NC + c
        base = s * rpt

        zero16 = jnp.zeros((_L,), jnp.float32)
        one16 = jnp.ones((_L,), jnp.float32)

        def init_row(r, carry):
            for k in range(d // _L):
                rows[r, pl.ds(k * _L, _L)] = zero16
            return carry

        lax.fori_loop(0, _CHUNK, init_row, 0)
        for k in range(_CHUNK // _L):
            ones1[pl.ds(k * _L, _L)] = one16

        def init_cbuf(i, carry):
            cbuf[pl.ds(i * _L, _L)] = zero16
            return carry

        lax.fori_loop(0, -(-rpt // _L), init_cbuf, 0)

        sizes = [_CHUNK] * (rpt // _CHUNK)
        if rpt % _CHUNK:
            sizes.append(rpt % _CHUNK)
        off = 0
        for sz in sizes:
            pltpu.sync_copy(rows.at[pl.ds(0, sz)],
                            s_sh.at[pl.ds(base + off, sz)])
            off += sz
        pltpu.sync_copy(cbuf.at[pl.ds(0, rpt)], c_sh.at[pl.ds(base, rpt)])

        pltpu.sync_copy(src_hbm.at[pl.ds(wid * cpt, cpt)], sall)
        pltpu.sync_copy(dst_hbm.at[pl.ds(wid * cpt, cpt)], dall)
        plsc.subcore_barrier()

        trip = (cpt * nworkers - wid + nworkers - 1) // nworkers

        def body(j, carry):
            for k in range(_CHUNK // _L):
                sidx[pl.ds(k * _L, _L)] = sall[j, pl.ds(k * _L, _L)]
                didx[pl.ds(k * _L, _L)] = dall[j, pl.ds(k * _L, _L)]
            pltpu.async_copy(x_hbm.at[sidx], rows, sem).wait()
            pltpu.sync_copy(rows, s_sh.at[didx], add=True)
            pltpu.sync_copy(ones1, c_sh.at[didx], add=True)
            return carry

        lax.fori_loop(0, trip, body, 0)
        plsc.subcore_barrier()

        off = 0
        for sz in sizes:
            r0 = base + off
            pltpu.sync_copy(s_sh.at[pl.ds(r0, sz)], rows.at[pl.ds(0, sz)])
            pltpu.sync_copy(rows.at[pl.ds(0, sz)], s_out.at[c, pl.ds(r0, sz)])
            off += sz
        pltpu.sync_copy(c_sh.at[pl.ds(base, rpt)], cbuf.at[pl.ds(0, rpt)])
        pltpu.sync_copy(cbuf.at[pl.ds(0, rpt)],
                        c_out.at[pl.ds(c * n_pad + base, rpt)])

    return sc_kernel(x, src2d, dst2d)


def _tc_combine_body(x_ref, s_ref, c_ref, w_ref, b_ref, o_ref):
    d = x_ref.shape[1]
    xb = x_ref[...]
    sb = s_ref[0] + s_ref[1]
    cnt = c_ref[0] + c_ref[1]
    w = w_ref[...]
    dn = (((1,), (1,)), ((), ()))
    t1 = lax.dot_general(xb, w[:, :d], dn,
                         preferred_element_type=jnp.float32,
                         precision=lax.Precision.HIGHEST)
    t2 = lax.dot_general(sb, w[:, d:], dn,
                         preferred_element_type=jnp.float32,
                         precision=lax.Precision.HIGHEST)
    inv = 1.0 / jnp.maximum(cnt, 1.0)
    o_ref[...] = jnp.where(cnt > 0.0, t1 + b_ref[...] + t2 * inv, 0.0)


def _tc_combine(x, s_parts, c_parts, W, b2d):
    n, d = x.shape
    blk = 1024
    grid = ((n + blk - 1) // blk,)
    return pl.pallas_call(
        _tc_combine_body,
        grid=grid,
        in_specs=[
            pl.BlockSpec((blk, d), lambda i: (i, 0)),
            pl.BlockSpec((_NC, blk, d), lambda i: (0, i, 0)),
            pl.BlockSpec((_NC, blk, 1), lambda i: (0, i, 0)),
            pl.BlockSpec((d, 2 * d), lambda i: (0, 0)),
            pl.BlockSpec((1, d), lambda i: (0, 0)),
        ],
        out_specs=pl.BlockSpec((blk, d), lambda i: (i, 0)),
        out_shape=jax.ShapeDtypeStruct((n, d), jnp.float32),
    )(x, s_parts, c_parts, W, b2d)


def kernel(x, edge_index, W, b):
    n, d = x.shape
    e = edge_index.shape[1]
    rpt = ((n + _NS - 1) // _NS + 7) // 8 * 8
    n_pad = rpt * _NS
    quantum = _CHUNK * _NC * _NS * _PAIR
    e_pad = (e + quantum - 1) // quantum * quantum
    src = jnp.pad(edge_index[0].astype(jnp.int32), (0, e_pad - e))
    dst = jnp.pad(edge_index[1].astype(jnp.int32), (0, e_pad - e),
                  constant_values=n)
    src2d = src.reshape(e_pad // _CHUNK, _CHUNK)
    dst2d = dst.reshape(e_pad // _CHUNK, _CHUNK)
    s_parts, c_flat = _sc_segment_sum(x, src2d, dst2d, n_pad)
    c_parts = c_flat.reshape(_NC, n_pad, 1)
    return _tc_combine(x, s_parts, c_parts, W, b.reshape(1, d))


# speedup vs baseline: 1.1393x; 1.1393x over previous
import functools

import jax
import jax.numpy as jnp
from jax import lax
from jax.experimental import pallas as pl
from jax.experimental.pallas import tpu as pltpu
from jax.experimental.pallas import tpu_sc as plsc

_CHUNK = 128
_PAIR = 2
_NC = 2
_NS = 16
_L = 16


def _sc_segment_sum(x, src2d, dst2d, n_pad):
    d = x.shape[1]
    nchunks = src2d.shape[0]
    rpt = n_pad // _NS
    nworkers = _NC * _NS
    cpt = nchunks // nworkers
    assert cpt * nworkers == nchunks

    mesh = plsc.VectorSubcoreMesh(core_axis_name="c", subcore_axis_name="s")

    @functools.partial(
        pl.kernel,
        out_type=(
            jax.ShapeDtypeStruct((_NC, n_pad, d), jnp.float32),
            jax.ShapeDtypeStruct((_NC * n_pad,), jnp.float32),
        ),
        mesh=mesh,
        scratch_types=[
            pltpu.VMEM((_CHUNK,), jnp.int32),
            pltpu.VMEM((_CHUNK,), jnp.int32),
            pltpu.VMEM((_CHUNK, d), jnp.float32),
            pltpu.VMEM((_CHUNK,), jnp.float32),
            pltpu.VMEM((-(-rpt // _L) * _L,), jnp.float32),
            pltpu.VMEM_SHARED((n_pad, d), jnp.float32),
            pltpu.VMEM_SHARED((n_pad,), jnp.float32),
            pltpu.SemaphoreType.DMA,
        ],
    )
    def sc_kernel(x_hbm, src_hbm, dst_hbm, s_out, c_out,
                  sidx, didx, rows, ones1, cbuf, s_sh, c_sh, sem):
        c = lax.axis_index("c")
        s = lax.axis_index("s")
        wid = s * _NC + c
        base = s * rpt

        zero16 = jnp.zeros((_L,), jnp.float32)
        one16 = jnp.ones((_L,), jnp.float32)

        def init_row(r, carry):
            for k in range(d // _L):
                rows[r, pl.ds(k * _L, _L)] = zero16
            return carry

        lax.fori_loop(0, _CHUNK, init_row, 0)
        for k in range(_CHUNK // _L):
            ones1[pl.ds(k * _L, _L)] = one16

        def init_cbuf(i, carry):
            cbuf[pl.ds(i * _L, _L)] = zero16
            return carry

        lax.fori_loop(0, -(-rpt // _L), init_cbuf, 0)

        sizes = [_CHUNK] * (rpt // _CHUNK)
        if rpt % _CHUNK:
            sizes.append(rpt % _CHUNK)
        off = 0
        for sz in sizes:
            pltpu.sync_copy(rows.at[pl.ds(0, sz)],
                            s_sh.at[pl.ds(base + off, sz)])
            off += sz
        pltpu.sync_copy(cbuf.at[pl.ds(0, rpt)], c_sh.at[pl.ds(base, rpt)])
        plsc.subcore_barrier()

        trip = (cpt * nworkers - wid + nworkers - 1) // nworkers

        def body(j, carry):
            cid = wid + j * nworkers
            pltpu.sync_copy(src_hbm.at[cid], sidx)
            pltpu.sync_copy(dst_hbm.at[cid], didx)
            pltpu.async_copy(x_hbm.at[sidx], rows, sem).wait()
            pltpu.sync_copy(rows, s_sh.at[didx], add=True)
            pltpu.sync_copy(ones1, c_sh.at[didx], add=True)
            return carry

        lax.fori_loop(0, trip, body, 0)
        plsc.subcore_barrier()

        off = 0
        for sz in sizes:
            r0 = base + off
            pltpu.sync_copy(s_sh.at[pl.ds(r0, sz)], rows.at[pl.ds(0, sz)])
            pltpu.sync_copy(rows.at[pl.ds(0, sz)], s_out.at[c, pl.ds(r0, sz)])
            off += sz
        pltpu.sync_copy(c_sh.at[pl.ds(base, rpt)], cbuf.at[pl.ds(0, rpt)])
        pltpu.sync_copy(cbuf.at[pl.ds(0, rpt)],
                        c_out.at[pl.ds(c * n_pad + base, rpt)])

    return sc_kernel(x, src2d, dst2d)


def _tc_combine_body(x_ref, s_ref, c_ref, w_ref, b_ref, o_ref):
    d = x_ref.shape[1]
    xb = x_ref[...]
    sb = s_ref[0] + s_ref[1]
    cnt = c_ref[0] + c_ref[1]
    w = w_ref[...]
    dn = (((1,), (1,)), ((), ()))
    t1 = lax.dot_general(xb, w[:, :d], dn,
                         preferred_element_type=jnp.float32,
                         precision=lax.Precision.HIGHEST)
    t2 = lax.dot_general(sb, w[:, d:], dn,
                         preferred_element_type=jnp.float32,
                         precision=lax.Precision.HIGHEST)
    inv = 1.0 / jnp.maximum(cnt, 1.0)
    o_ref[...] = jnp.where(cnt > 0.0, t1 + b_ref[...] + t2 * inv, 0.0)


def _tc_combine(x, s_parts, c_parts, W, b2d):
    n, d = x.shape
    blk = 1024
    grid = ((n + blk - 1) // blk,)
    return pl.pallas_call(
        _tc_combine_body,
        grid=grid,
        in_specs=[
            pl.BlockSpec((blk, d), lambda i: (i, 0)),
            pl.BlockSpec((_NC, blk, d), lambda i: (0, i, 0)),
            pl.BlockSpec((_NC, blk, 1), lambda i: (0, i, 0)),
            pl.BlockSpec((d, 2 * d), lambda i: (0, 0)),
            pl.BlockSpec((1, d), lambda i: (0, 0)),
        ],
        out_specs=pl.BlockSpec((blk, d), lambda i: (i, 0)),
        out_shape=jax.ShapeDtypeStruct((n, d), jnp.float32),
    )(x, s_parts, c_parts, W, b2d)


def kernel(x, edge_index, W, b):
    n, d = x.shape
    e = edge_index.shape[1]
    rpt = ((n + _NS - 1) // _NS + 7) // 8 * 8
    n_pad = rpt * _NS
    quantum = _CHUNK * _NC * _NS * _PAIR
    e_pad = (e + quantum - 1) // quantum * quantum
    src = jnp.pad(edge_index[0].astype(jnp.int32), (0, e_pad - e))
    dst = jnp.pad(edge_index[1].astype(jnp.int32), (0, e_pad - e),
                  constant_values=n)
    src2d = src.reshape(e_pad // _CHUNK, _CHUNK)
    dst2d = dst.reshape(e_pad // _CHUNK, _CHUNK)
    s_parts, c_flat = _sc_segment_sum(x, src2d, dst2d, n_pad)
    c_parts = c_flat.reshape(_NC, n_pad, 1)
    return _tc_combine(x, s_parts, c_parts, W, b.reshape(1, d))


# speedup vs baseline: 1.1394x; 1.0001x over previous
import functools

import jax
import jax.numpy as jnp
from jax import lax
from jax.experimental import pallas as pl
from jax.experimental.pallas import tpu as pltpu
from jax.experimental.pallas import tpu_sc as plsc

_CHUNK = 128
_PAIR = 2
_NC = 2
_NS = 16
_L = 16


def _sc_segment_sum(x, src2d, dst2d, n_pad):
    d = x.shape[1]
    nchunks = src2d.shape[0]
    rpt = n_pad // _NS
    nworkers = _NC * _NS
    cpt = nchunks // nworkers
    assert cpt * nworkers == nchunks

    mesh = plsc.VectorSubcoreMesh(core_axis_name="c", subcore_axis_name="s")

    @functools.partial(
        pl.kernel,
        out_type=(
            jax.ShapeDtypeStruct((_NC, n_pad, d), jnp.float32),
            jax.ShapeDtypeStruct((_NC * n_pad,), jnp.float32),
        ),
        mesh=mesh,
        scratch_types=[
            pltpu.VMEM((_CHUNK,), jnp.int32),
            pltpu.VMEM((1, _CHUNK), jnp.int32),
            pltpu.VMEM((_CHUNK, d), jnp.float32),
            pltpu.VMEM((_CHUNK,), jnp.float32),
            pltpu.VMEM((-(-rpt // _L) * _L,), jnp.float32),
            pltpu.VMEM_SHARED((n_pad, d), jnp.float32),
            pltpu.VMEM_SHARED((n_pad,), jnp.float32),
            pltpu.SemaphoreType.DMA,
        ],
    )
    def sc_kernel(x_hbm, src_hbm, dst_hbm, s_out, c_out,
                  sidx, didx, rows, ones1, cbuf, s_sh, c_sh, sem):
        c = lax.axis_index("c")
        s = lax.axis_index("s")
        wid = s * _NC + c
        base = s * rpt

        zero16 = jnp.zeros((_L,), jnp.float32)
        one16 = jnp.ones((_L,), jnp.float32)

        def init_row(r, carry):
            for k in range(d // _L):
                rows[r, pl.ds(k * _L, _L)] = zero16
            return carry

        lax.fori_loop(0, _CHUNK, init_row, 0)
        for k in range(_CHUNK // _L):
            ones1[pl.ds(k * _L, _L)] = one16

        def init_cbuf(i, carry):
            cbuf[pl.ds(i * _L, _L)] = zero16
            return carry

        lax.fori_loop(0, -(-rpt // _L), init_cbuf, 0)

        sizes = [_CHUNK] * (rpt // _CHUNK)
        if rpt % _CHUNK:
            sizes.append(rpt % _CHUNK)
        off = 0
        for sz in sizes:
            pltpu.sync_copy(rows.at[pl.ds(0, sz)],
                            s_sh.at[pl.ds(base + off, sz)])
            off += sz
        pltpu.sync_copy(cbuf.at[pl.ds(0, rpt)], c_sh.at[pl.ds(base, rpt)])
        plsc.subcore_barrier()

        trip = (cpt * nworkers - wid + nworkers - 1) // nworkers

        def body(j, carry):
            cid = wid + j * nworkers
            pltpu.sync_copy(src_hbm.at[cid], sidx)
            pltpu.sync_copy(dst_hbm.at[cid], didx.at[0])
            pltpu.async_copy(x_hbm.at[sidx], rows, sem).wait()
            pltpu.sync_copy(rows, s_sh.at[didx.at[0]], add=True)
            pltpu.sync_copy(ones1, c_sh.at[didx.at[0]], add=True)
            return carry

        lax.fori_loop(0, trip, body, 0)
        plsc.subcore_barrier()

        off = 0
        for sz in sizes:
            r0 = base + off
            pltpu.sync_copy(s_sh.at[pl.ds(r0, sz)], rows.at[pl.ds(0, sz)])
            pltpu.sync_copy(rows.at[pl.ds(0, sz)], s_out.at[c, pl.ds(r0, sz)])
            off += sz
        pltpu.sync_copy(c_sh.at[pl.ds(base, rpt)], cbuf.at[pl.ds(0, rpt)])
        pltpu.sync_copy(cbuf.at[pl.ds(0, rpt)],
                        c_out.at[pl.ds(c * n_pad + base, rpt)])

    return sc_kernel(x, src2d, dst2d)


def _tc_combine_body(x_ref, s_ref, c_ref, w_ref, b_ref, o_ref):
    d = x_ref.shape[1]
    xb = x_ref[...]
    sb = s_ref[0] + s_ref[1]
    cnt = c_ref[0] + c_ref[1]
    w = w_ref[...]
    dn = (((1,), (1,)), ((), ()))
    t1 = lax.dot_general(xb, w[:, :d], dn,
                         preferred_element_type=jnp.float32,
                         precision=lax.Precision.HIGHEST)
    t2 = lax.dot_general(sb, w[:, d:], dn,
                         preferred_element_type=jnp.float32,
                         precision=lax.Precision.HIGHEST)
    inv = 1.0 / jnp.maximum(cnt, 1.0)
    o_ref[...] = jnp.where(cnt > 0.0, t1 + b_ref[...] + t2 * inv, 0.0)


def _tc_combine(x, s_parts, c_parts, W, b2d):
    n, d = x.shape
    blk = 1024
    grid = ((n + blk - 1) // blk,)
    return pl.pallas_call(
        _tc_combine_body,
        grid=grid,
        in_specs=[
            pl.BlockSpec((blk, d), lambda i: (i, 0)),
            pl.BlockSpec((_NC, blk, d), lambda i: (0, i, 0)),
            pl.BlockSpec((_NC, blk, 1), lambda i: (0, i, 0)),
            pl.BlockSpec((d, 2 * d), lambda i: (0, 0)),
            pl.BlockSpec((1, d), lambda i: (0, 0)),
        ],
        out_specs=pl.BlockSpec((blk, d), lambda i: (i, 0)),
        out_shape=jax.ShapeDtypeStruct((n, d), jnp.float32),
    )(x, s_parts, c_parts, W, b2d)


def kernel(x, edge_index, W, b):
    n, d = x.shape
    e = edge_index.shape[1]
    rpt = ((n + _NS - 1) // _NS + 7) // 8 * 8
    n_pad = rpt * _NS
    quantum = _CHUNK * _NC * _NS * _PAIR
    e_pad = (e + quantum - 1) // quantum * quantum
    src = jnp.pad(edge_index[0].astype(jnp.int32), (0, e_pad - e))
    dst = jnp.pad(edge_index[1].astype(jnp.int32), (0, e_pad - e),
                  constant_values=n)
    src2d = src.reshape(e_pad // _CHUNK, _CHUNK)
    dst2d = dst.reshape(e_pad // _CHUNK, _CHUNK)
    s_parts, c_flat = _sc_segment_sum(x, src2d, dst2d, n_pad)
    c_parts = c_flat.reshape(_NC, n_pad, 1)
    return _tc_combine(x, s_parts, c_parts, W, b.reshape(1, d))


# speedup vs baseline: 2.1550x; 1.8912x over previous
import functools

import jax
import jax.numpy as jnp
from jax import lax
from jax.experimental import pallas as pl
from jax.experimental.pallas import tpu as pltpu
from jax.experimental.pallas import tpu_sc as plsc

_CHUNK = 128
_PAIR = 2
_NC = 2
_NS = 16
_L = 16


def _sc_segment_sum(x, src2d, dst2d, n_pad):
    d = x.shape[1]
    nchunks = src2d.shape[0]
    rpt = n_pad // _NS
    nworkers = _NC * _NS
    cpt = -(-nchunks // nworkers)

    mesh = plsc.VectorSubcoreMesh(core_axis_name="c", subcore_axis_name="s")

    @functools.partial(
        pl.kernel,
        out_type=(
            jax.ShapeDtypeStruct((_NC, n_pad, d), jnp.float32),
            jax.ShapeDtypeStruct((_NC * n_pad,), jnp.float32),
        ),
        mesh=mesh,
        scratch_types=[
            pltpu.VMEM((_CHUNK,), jnp.int32),
            pltpu.VMEM((1, _CHUNK), jnp.int32),
            pltpu.VMEM((_CHUNK, d), jnp.float32),
            pltpu.VMEM((_CHUNK,), jnp.float32),
            pltpu.VMEM((-(-rpt // _L) * _L,), jnp.float32),
            pltpu.VMEM_SHARED((n_pad, d), jnp.float32),
            pltpu.VMEM_SHARED((n_pad,), jnp.float32),
            pltpu.SemaphoreType.DMA,
        ],
    )
    def sc_kernel(x_hbm, src_hbm, dst_hbm, s_out, c_out,
                  sidx, didx, rows, ones1, cbuf, s_sh, c_sh, sem):
        c = lax.axis_index("c")
        s = lax.axis_index("s")
        wid = s * _NC + c
        base = s * rpt

        zero16 = jnp.zeros((_L,), jnp.float32)
        one16 = jnp.ones((_L,), jnp.float32)

        def init_row(r, carry):
            for k in range(d // _L):
                rows[r, pl.ds(k * _L, _L)] = zero16
            return carry

        lax.fori_loop(0, _CHUNK, init_row, 0)
        for k in range(_CHUNK // _L):
            ones1[pl.ds(k * _L, _L)] = one16

        def init_cbuf(i, carry):
            cbuf[pl.ds(i * _L, _L)] = zero16
            return carry

        lax.fori_loop(0, -(-rpt // _L), init_cbuf, 0)

        sizes = [_CHUNK] * (rpt // _CHUNK)
        if rpt % _CHUNK:
            sizes.append(rpt % _CHUNK)
        off = 0
        for sz in sizes:
            pltpu.sync_copy(rows.at[pl.ds(0, sz)],
                            s_sh.at[pl.ds(base + off, sz)])
            off += sz
        pltpu.sync_copy(cbuf.at[pl.ds(0, rpt)], c_sh.at[pl.ds(base, rpt)])
        plsc.subcore_barrier()

        trip = (nchunks - wid + nworkers - 1) // nworkers

        def body(j, carry):
            cid = wid + j * nworkers
            pltpu.sync_copy(src_hbm.at[cid], sidx)
            pltpu.sync_copy(dst_hbm.at[cid], didx.at[0])
            pltpu.async_copy(x_hbm.at[sidx], rows, sem).wait()
            pltpu.sync_copy(rows, s_sh.at[didx.at[0]], add=True)
            pltpu.sync_copy(ones1, c_sh.at[didx.at[0]], add=True)
            return carry

        lax.fori_loop(0, trip, body, 0)
        plsc.subcore_barrier()

        off = 0
        for sz in sizes:
            r0 = base + off
            pltpu.sync_copy(s_sh.at[pl.ds(r0, sz)], rows.at[pl.ds(0, sz)])
            pltpu.sync_copy(rows.at[pl.ds(0, sz)], s_out.at[c, pl.ds(r0, sz)])
            off += sz
        pltpu.sync_copy(c_sh.at[pl.ds(base, rpt)], cbuf.at[pl.ds(0, rpt)])
        pltpu.sync_copy(cbuf.at[pl.ds(0, rpt)],
                        c_out.at[pl.ds(c * n_pad + base, rpt)])

    return sc_kernel(x, src2d, dst2d)


def _tc_combine_body(x_ref, s_ref, c_ref, w_ref, b_ref, o_ref):
    d = x_ref.shape[1]
    xb = x_ref[...]
    sb = s_ref[0] + s_ref[1]
    cnt = c_ref[0] + c_ref[1]
    w = w_ref[...]
    dn = (((1,), (1,)), ((), ()))
    t1 = lax.dot_general(xb, w[:, :d], dn,
                         preferred_element_type=jnp.float32,
                         precision=lax.Precision.HIGHEST)
    t2 = lax.dot_general(sb, w[:, d:], dn,
                         preferred_element_type=jnp.float32,
                         precision=lax.Precision.HIGHEST)
    inv = 1.0 / jnp.maximum(cnt, 1.0)
    o_ref[...] = jnp.where(cnt > 0.0, t1 + b_ref[...] + t2 * inv, 0.0)


def _tc_combine(x, s_parts, c_parts, W, b2d):
    n, d = x.shape
    blk = 1024
    grid = ((n + blk - 1) // blk,)
    return pl.pallas_call(
        _tc_combine_body,
        grid=grid,
        in_specs=[
            pl.BlockSpec((blk, d), lambda i: (i, 0)),
            pl.BlockSpec((_NC, blk, d), lambda i: (0, i, 0)),
            pl.BlockSpec((_NC, blk, 1), lambda i: (0, i, 0)),
            pl.BlockSpec((d, 2 * d), lambda i: (0, 0)),
            pl.BlockSpec((1, d), lambda i: (0, 0)),
        ],
        out_specs=pl.BlockSpec((blk, d), lambda i: (i, 0)),
        out_shape=jax.ShapeDtypeStruct((n, d), jnp.float32),
    )(x, s_parts, c_parts, W, b2d)


def kernel(x, edge_index, W, b):
    n, d = x.shape
    e = edge_index.shape[1]
    rpt = ((n + _NS - 1) // _NS + 7) // 8 * 8
    n_pad = rpt * _NS
    e_pad = (e + _CHUNK - 1) // _CHUNK * _CHUNK
    src = jnp.pad(edge_index[0].astype(jnp.int32), (0, e_pad - e))
    dst = jnp.pad(edge_index[1].astype(jnp.int32), (0, e_pad - e),
                  constant_values=n)
    src2d = src.reshape(e_pad // _CHUNK, _CHUNK)
    dst2d = dst.reshape(e_pad // _CHUNK, _CHUNK)
    s_parts, c_flat = _sc_segment_sum(x, src2d, dst2d, n_pad)
    c_parts = c_flat.reshape(_NC, n_pad, 1)
    return _tc_combine(x, s_parts, c_parts, W, b.reshape(1, d))


# speedup vs baseline: 2.1551x; 1.0001x over previous
import functools

import jax
import jax.numpy as jnp
from jax import lax
from jax.experimental import pallas as pl
from jax.experimental.pallas import tpu as pltpu
from jax.experimental.pallas import tpu_sc as plsc

_CHUNK = 128
_NC = 2
_NS = 16
_L = 16


def _sc_segment_sum(x, src2d, dst2d, n_pad):
    d = x.shape[1]
    nchunks = src2d.shape[0]
    rpt = n_pad // _NS
    nworkers = _NC * _NS

    mesh = plsc.VectorSubcoreMesh(core_axis_name="c", subcore_axis_name="s")

    @functools.partial(
        pl.kernel,
        out_type=(
            jax.ShapeDtypeStruct((_NC, n_pad, d), jnp.float32),
            jax.ShapeDtypeStruct((_NC * n_pad,), jnp.float32),
        ),
        mesh=mesh,
        scratch_types=[
            pltpu.VMEM((_CHUNK,), jnp.int32),
            pltpu.VMEM((1, _CHUNK), jnp.int32),
            pltpu.VMEM((_CHUNK, d), jnp.float32),
            pltpu.VMEM((_CHUNK,), jnp.float32),
            pltpu.VMEM((-(-rpt // _L) * _L,), jnp.float32),
            pltpu.VMEM_SHARED((n_pad, d), jnp.float32),
            pltpu.VMEM_SHARED((n_pad,), jnp.float32),
            pltpu.SemaphoreType.DMA,
        ],
    )
    def sc_kernel(x_hbm, src_hbm, dst_hbm, s_out, c_out,
                  sidx, didx, rows, ones1, cbuf, s_sh, c_sh, sem):
        c = lax.axis_index("c")
        s = lax.axis_index("s")
        wid = s * _NC + c
        base = s * rpt

        zero16 = jnp.zeros((_L,), jnp.float32)
        one16 = jnp.ones((_L,), jnp.float32)

        def init_row(r, carry):
            for k in range(d // _L):
                rows[r, pl.ds(k * _L, _L)] = zero16
            return carry

        lax.fori_loop(0, _CHUNK, init_row, 0)
        for k in range(_CHUNK // _L):
            ones1[pl.ds(k * _L, _L)] = one16

        def init_cbuf(i, carry):
            cbuf[pl.ds(i * _L, _L)] = zero16
            return carry

        lax.fori_loop(0, -(-rpt // _L), init_cbuf, 0)

        sizes = [_CHUNK] * (rpt // _CHUNK)
        if rpt % _CHUNK:
            sizes.append(rpt % _CHUNK)
        off = 0
        for sz in sizes:
            pltpu.sync_copy(rows.at[pl.ds(0, sz)],
                            s_sh.at[pl.ds(base + off, sz)])
            off += sz
        pltpu.sync_copy(cbuf.at[pl.ds(0, rpt)], c_sh.at[pl.ds(base, rpt)])
        plsc.subcore_barrier()

        trip = (nchunks - wid + nworkers - 1) // nworkers

        def body(j, carry):
            cid = wid + j * nworkers
            pltpu.sync_copy(src_hbm.at[cid], sidx)
            pltpu.sync_copy(dst_hbm.at[cid], didx.at[0])
            pltpu.async_copy(x_hbm.at[sidx], rows, sem).wait()
            pltpu.sync_copy(rows, s_sh.at[didx.at[0]], add=True)
            pltpu.sync_copy(ones1, c_sh.at[didx.at[0]], add=True)
            return carry

        lax.fori_loop(0, trip, body, 0)
        plsc.subcore_barrier()

        off = 0
        for sz in sizes:
            r0 = base + off
            pltpu.sync_copy(s_sh.at[pl.ds(r0, sz)], rows.at[pl.ds(0, sz)])
            pltpu.sync_copy(rows.at[pl.ds(0, sz)], s_out.at[c, pl.ds(r0, sz)])
            off += sz
        pltpu.sync_copy(c_sh.at[pl.ds(base, rpt)], cbuf.at[pl.ds(0, rpt)])
        pltpu.sync_copy(cbuf.at[pl.ds(0, rpt)],
                        c_out.at[pl.ds(c * n_pad + base, rpt)])

    return sc_kernel(x, src2d, dst2d)


def _tc_combine_body(x_ref, s_ref, c_ref, w_ref, b_ref, o_ref):
    d = x_ref.shape[1]
    xb = x_ref[...]
    sb = s_ref[0] + s_ref[1]
    cnt = c_ref[0] + c_ref[1]
    w = w_ref[...]
    dn = (((1,), (1,)), ((), ()))
    t1 = lax.dot_general(xb, w[:, :d], dn,
                         preferred_element_type=jnp.float32,
                         precision=lax.Precision.HIGHEST)
    t2 = lax.dot_general(sb, w[:, d:], dn,
                         preferred_element_type=jnp.float32,
                         precision=lax.Precision.HIGHEST)
    inv = 1.0 / jnp.maximum(cnt, 1.0)
    o_ref[...] = jnp.where(cnt > 0.0, t1 + b_ref[...] + t2 * inv, 0.0)


def _tc_combine(x, s_parts, c_parts, W, b2d):
    n, d = x.shape
    blk = 1024
    grid = ((n + blk - 1) // blk,)
    return pl.pallas_call(
        _tc_combine_body,
        grid=grid,
        in_specs=[
            pl.BlockSpec((blk, d), lambda i: (i, 0)),
            pl.BlockSpec((_NC, blk, d), lambda i: (0, i, 0)),
            pl.BlockSpec((_NC, blk, 1), lambda i: (0, i, 0)),
            pl.BlockSpec((d, 2 * d), lambda i: (0, 0)),
            pl.BlockSpec((1, d), lambda i: (0, 0)),
        ],
        out_specs=pl.BlockSpec((blk, d), lambda i: (i, 0)),
        out_shape=jax.ShapeDtypeStruct((n, d), jnp.float32),
    )(x, s_parts, c_parts, W, b2d)


def kernel(x, edge_index, W, b):
    n, d = x.shape
    e = edge_index.shape[1]
    rpt = ((n + _NS - 1) // _NS + 7) // 8 * 8
    n_pad = rpt * _NS
    e_pad = (e + _CHUNK - 1) // _CHUNK * _CHUNK
    src = jnp.pad(edge_index[0].astype(jnp.int32), (0, e_pad - e))
    dst = jnp.pad(edge_index[1].astype(jnp.int32), (0, e_pad - e),
                  constant_values=n)
    src2d = src.reshape(e_pad // _CHUNK, _CHUNK)
    dst2d = dst.reshape(e_pad // _CHUNK, _CHUNK)
    s_parts, c_flat = _sc_segment_sum(x, src2d, dst2d, n_pad)
    c_parts = c_flat.reshape(_NC, n_pad, 1)
    return _tc_combine(x, s_parts, c_parts, W, b.reshape(1, d))
